# baseline (speedup 1.0000x reference)
import jax
import jax.numpy as jnp
from jax.experimental import pallas as pl

B = 16384
D = 128
R = 1024
NB = B // R


def _body(x_ref, gu_ref, gi_ref, xui_ref):
    gu = x_ref[0]
    gi = x_ref[1]
    gu_ref[...] = gu
    gi_ref[...] = gi
    xui_ref[...] = jnp.sum(gu * gi, axis=1).reshape(R // 128, 128)


def kernel(inputs):
    gu_out, gi_out, xui2d = pl.pallas_call(
        _body,
        grid=(NB,),
        in_specs=[pl.BlockSpec((2, R, D), lambda i: (0, i, 0))],
        out_specs=[
            pl.BlockSpec((R, D), lambda i: (i, 0)),
            pl.BlockSpec((R, D), lambda i: (i, 0)),
            pl.BlockSpec((R // 128, 128), lambda i: (i, 0)),
        ],
        out_shape=[
            jax.ShapeDtypeStruct((B, D), jnp.float32),
            jax.ShapeDtypeStruct((B, D), jnp.float32),
            jax.ShapeDtypeStruct((B // 128, 128), jnp.float32),
        ],
    )(inputs)
    return (xui2d.reshape(B), gu_out, gi_out)


# speedup vs baseline: 1.4211x; 1.4211x over previous
import jax
import jax.numpy as jnp
from jax.experimental import pallas as pl

B = 16384
D = 128
R = 4096
NB = B // R


def _body(x_ref, gu_ref, gi_ref, xui_ref):
    gu = x_ref[0]
    gi = x_ref[1]
    gu_ref[...] = gu
    gi_ref[...] = gi
    xui_ref[...] = jnp.sum(gu * gi, axis=1).reshape(R // 128, 128)


def kernel(inputs):
    gu_out, gi_out, xui2d = pl.pallas_call(
        _body,
        grid=(NB,),
        in_specs=[pl.BlockSpec((2, R, D), lambda i: (0, i, 0))],
        out_specs=[
            pl.BlockSpec((R, D), lambda i: (i, 0)),
            pl.BlockSpec((R, D), lambda i: (i, 0)),
            pl.BlockSpec((R // 128, 128), lambda i: (i, 0)),
        ],
        out_shape=[
            jax.ShapeDtypeStruct((B, D), jnp.float32),
            jax.ShapeDtypeStruct((B, D), jnp.float32),
            jax.ShapeDtypeStruct((B // 128, 128), jnp.float32),
        ],
    )(inputs)
    return (xui2d.reshape(B), gu_out, gi_out)


# speedup vs baseline: 1.6102x; 1.1331x over previous
import jax
import jax.numpy as jnp
from jax.experimental import pallas as pl

B = 16384
D = 128
R = 8192
NB = B // R


def _body(x_ref, gu_ref, gi_ref, xui_ref):
    gu = x_ref[0]
    gi = x_ref[1]
    gu_ref[...] = gu
    gi_ref[...] = gi
    xui_ref[...] = jnp.sum(gu * gi, axis=1).reshape(R // 128, 128)


def kernel(inputs):
    gu_out, gi_out, xui2d = pl.pallas_call(
        _body,
        grid=(NB,),
        in_specs=[pl.BlockSpec((2, R, D), lambda i: (0, i, 0))],
        out_specs=[
            pl.BlockSpec((R, D), lambda i: (i, 0)),
            pl.BlockSpec((R, D), lambda i: (i, 0)),
            pl.BlockSpec((R // 128, 128), lambda i: (i, 0)),
        ],
        out_shape=[
            jax.ShapeDtypeStruct((B, D), jnp.float32),
            jax.ShapeDtypeStruct((B, D), jnp.float32),
            jax.ShapeDtypeStruct((B // 128, 128), jnp.float32),
        ],
    )(inputs)
    return (xui2d.reshape(B), gu_out, gi_out)
